# Initial kernel scaffold; baseline (speedup 1.0000x reference)
import jax
import jax.numpy as jnp
from jax.experimental import pallas as pl


def kernel(x, edge_index, W1l, W1r, b1, W2l, W2r, b2, W3l, W3r, b3):
    raise NotImplementedError("write your pallas kernel here")


# speedup vs baseline: 2.7604x; 2.7604x over previous
import functools

import jax
import jax.numpy as jnp
from jax import lax
from jax.experimental import pallas as pl
from jax.experimental.pallas import tpu as pltpu
from jax.experimental.pallas import tpu_sc as plsc

N = 10000
E = 160000
LANES = 128
CHUNK = 128
NS = 16
EPC = -(-E // (NS * CHUNK))
EPT = EPC * CHUNK
E_PAD = EPT * NS
ACC_ROWS = 10112
ZROWS = ACC_ROWS // NS
ZCHUNKS = (128, 128, 128, 128, 120)
MT = 2000


def _sc_mesh():
    return plsc.VectorSubcoreMesh(core_axis_name="c", subcore_axis_name="s")


@functools.lru_cache(maxsize=None)
def _segsum(nb):
    bpc = nb // 2

    def body(*refs):
        fs = refs[:nb]
        src_hbm, dst_hbm, z_hbm = refs[nb:nb + 3]
        outs = refs[nb + 3:nb + 3 + nb]
        src_v, dst_v, rows_v, zrows_v, acc, sem = refs[nb + 3 + nb:]
        cid = lax.axis_index("c")
        sid = lax.axis_index("s")
        pltpu.sync_copy(z_hbm, zrows_v)
        for b in range(nb):
            @pl.when(cid == b // bpc)
            def _(b=b):
                r0 = sid * ZROWS
                for nz in ZCHUNKS:
                    pltpu.sync_copy(zrows_v.at[pl.ds(0, nz)],
                                    acc.at[pl.ds(r0, nz)])
                    r0 = r0 + nz
                plsc.subcore_barrier()

                def edge_chunk(g, carry):
                    off = sid * EPT + g * CHUNK
                    pltpu.sync_copy(src_hbm.at[pl.ds(off, CHUNK)], src_v)
                    pltpu.sync_copy(dst_hbm.at[pl.ds(off, CHUNK)], dst_v)
                    pltpu.async_copy(fs[b].at[src_v], rows_v, sem).wait()
                    pltpu.sync_copy(rows_v, acc.at[dst_v], add=True)
                    return carry

                lax.fori_loop(0, EPC, edge_chunk, 0)
                plsc.subcore_barrier()
                r0 = sid * ZROWS
                for nz in ZCHUNKS:
                    pltpu.sync_copy(acc.at[pl.ds(r0, nz)],
                                    rows_v.at[pl.ds(0, nz)])
                    pltpu.sync_copy(rows_v.at[pl.ds(0, nz)],
                                    outs[b].at[pl.ds(r0, nz)])
                    r0 = r0 + nz

    return pl.kernel(
        body,
        out_type=[jax.ShapeDtypeStruct((ACC_ROWS, LANES), jnp.float32)
                  for _ in range(nb)],
        mesh=_sc_mesh(),
        scratch_types=[
            pltpu.VMEM((CHUNK,), jnp.int32),
            pltpu.VMEM((CHUNK,), jnp.int32),
            pltpu.VMEM((CHUNK, LANES), jnp.float32),
            pltpu.VMEM((128, LANES), jnp.float32),
            pltpu.VMEM_SHARED((ACC_ROWS, LANES), jnp.float32),
            pltpu.SemaphoreType.DMA,
        ],
        name=f"sage_segsum_{nb}",
    )


def _count_body(dst_hbm, z_hbm, ones_hbm, out_hbm, dst_v, ones_v, zrows_v,
                rows_v, cacc):
    cid = lax.axis_index("c")
    sid = lax.axis_index("s")

    @pl.when(cid == 0)
    def _():
        pltpu.sync_copy(z_hbm, zrows_v)
        pltpu.sync_copy(ones_hbm, ones_v)
        r0 = sid * ZROWS
        for nz in ZCHUNKS:
            pltpu.sync_copy(zrows_v.at[pl.ds(0, nz)], cacc.at[pl.ds(r0, nz)])
            r0 = r0 + nz
        plsc.subcore_barrier()

        def edge_chunk(g, carry):
            off = sid * EPT + g * CHUNK
            pltpu.sync_copy(dst_hbm.at[pl.ds(off, CHUNK)], dst_v)
            pltpu.sync_copy(ones_v, cacc.at[dst_v], add=True)
            return carry

        lax.fori_loop(0, EPC, edge_chunk, 0)
        plsc.subcore_barrier()
        r0 = sid * ZROWS
        for nz in ZCHUNKS:
            pltpu.sync_copy(cacc.at[pl.ds(r0, nz)], rows_v.at[pl.ds(0, nz)])
            pltpu.sync_copy(rows_v.at[pl.ds(0, nz)], out_hbm.at[pl.ds(r0, nz)])
            r0 = r0 + nz


@functools.lru_cache(maxsize=None)
def _count_kernel():
    return pl.kernel(
        _count_body,
        out_type=jax.ShapeDtypeStruct((ACC_ROWS, LANES), jnp.float32),
        mesh=_sc_mesh(),
        scratch_types=[
            pltpu.VMEM((CHUNK,), jnp.int32),
            pltpu.VMEM((CHUNK, LANES), jnp.float32),
            pltpu.VMEM((128, LANES), jnp.float32),
            pltpu.VMEM((128, LANES), jnp.float32),
            pltpu.VMEM_SHARED((ACC_ROWS, LANES), jnp.float32),
        ],
        name="sage_counts",
    )


@functools.lru_cache(maxsize=None)
def _tc_layer(d_in, d_out, relu):
    def body(agg_ref, cnt_ref, x_ref, wl_ref, wr_ref, b_ref, out_ref):
        inv = 1.0 / jnp.maximum(cnt_ref[...][:, 0:1], 1.0)
        mean = agg_ref[...] * inv
        acc = jnp.dot(mean, wl_ref[...], preferred_element_type=jnp.float32)
        acc = acc + jnp.dot(x_ref[...], wr_ref[...],
                            preferred_element_type=jnp.float32)
        acc = acc + b_ref[...]
        out_ref[...] = jnp.maximum(acc, 0.0) if relu else acc

    return pl.pallas_call(
        body,
        grid=(N // MT,),
        in_specs=[
            pl.BlockSpec((MT, d_in), lambda i: (i, 0)),
            pl.BlockSpec((MT, 16), lambda i: (i, 0)),
            pl.BlockSpec((MT, d_in), lambda i: (i, 0)),
            pl.BlockSpec((d_in, d_out), lambda i: (0, 0)),
            pl.BlockSpec((d_in, d_out), lambda i: (0, 0)),
            pl.BlockSpec((1, d_out), lambda i: (0, 0)),
        ],
        out_specs=pl.BlockSpec((MT, d_out), lambda i: (i, 0)),
        out_shape=jax.ShapeDtypeStruct((N, d_out), jnp.float32),
    )


def kernel(x, edge_index, W1l, W1r, b1, W2l, W2r, b2, W3l, W3r, b3):
    src = edge_index[0].astype(jnp.int32)
    dst = edge_index[1].astype(jnp.int32)
    pad = E_PAD - E
    src = jnp.concatenate([src, jnp.zeros((pad,), jnp.int32)])
    dst = jnp.concatenate([dst, jnp.full((pad,), N, jnp.int32)])
    z128 = jnp.zeros((128, LANES), jnp.float32)
    ones128 = jnp.ones((CHUNK, LANES), jnp.float32)
    cnt16 = _count_kernel()(dst, z128, ones128)[:N, :16]

    h = x
    for wl, wr, b, relu in ((W1l, W1r, b1, True), (W2l, W2r, b2, True),
                            (W3l, W3r, b3, False)):
        d = h.shape[1]
        nb = d // LANES
        blocks = [lax.slice_in_dim(h, i * LANES, (i + 1) * LANES, axis=1)
                  for i in range(nb)]
        aggs = _segsum(nb)(*blocks, src, dst, z128)
        agg = jnp.concatenate([a[:N] for a in aggs], axis=1)
        h = _tc_layer(d, wl.shape[1], relu)(agg, cnt16, h, wl, wr,
                                            b.reshape(1, -1))
    return h


# speedup vs baseline: 2.8315x; 1.0257x over previous
import functools

import jax
import jax.numpy as jnp
from jax import lax
from jax.experimental import pallas as pl
from jax.experimental.pallas import tpu as pltpu
from jax.experimental.pallas import tpu_sc as plsc

N = 10000
E = 160000
LANES = 128
CHUNK = 128
NS = 16
EPC = 80
EPT = EPC * CHUNK
E_PAD = EPT * NS
ACC_ROWS = 10112
ZROWS = ACC_ROWS // NS
ZCHUNKS = (128, 128, 128, 128, 120)
MT = 2000


def _sc_mesh():
    return plsc.VectorSubcoreMesh(core_axis_name="c", subcore_axis_name="s")


@functools.lru_cache(maxsize=None)
def _segsum(nb):
    bpc = nb // 2

    def body(*refs):
        fs = refs[:nb]
        src_hbm, dst_hbm, z_hbm = refs[nb:nb + 3]
        outs = refs[nb + 3:nb + 3 + nb]
        (src_a, dst_a, src_b, dst_b, rows0, rows1, zrows_v, acc,
         sem0, sem1) = refs[nb + 3 + nb:]
        cid = lax.axis_index("c")
        sid = lax.axis_index("s")
        pltpu.sync_copy(z_hbm, zrows_v)
        for b in range(nb):
            @pl.when(cid == b // bpc)
            def _(b=b):
                r0 = sid * ZROWS
                for nz in ZCHUNKS:
                    pltpu.sync_copy(zrows_v.at[pl.ds(0, nz)],
                                    acc.at[pl.ds(r0, nz)])
                    r0 = r0 + nz
                plsc.subcore_barrier()

                base = sid * EPT
                pltpu.sync_copy(src_hbm.at[pl.ds(base, CHUNK)], src_a)
                pltpu.sync_copy(dst_hbm.at[pl.ds(base, CHUNK)], dst_a)
                pltpu.async_copy(fs[b].at[src_a], rows0, sem0)

                def pair(p, carry):
                    off = base + (2 * p + 1) * CHUNK
                    pltpu.sync_copy(src_hbm.at[pl.ds(off, CHUNK)], src_b)
                    pltpu.sync_copy(dst_hbm.at[pl.ds(off, CHUNK)], dst_b)
                    pltpu.async_copy(fs[b].at[src_b], rows1, sem1)
                    pltpu.make_async_copy(fs[b].at[pl.ds(0, CHUNK)], rows0, sem0).wait()
                    pltpu.sync_copy(rows0, acc.at[dst_a], add=True)
                    off = base + ((2 * p + 2) % EPC) * CHUNK
                    pltpu.sync_copy(src_hbm.at[pl.ds(off, CHUNK)], src_a)
                    pltpu.sync_copy(dst_hbm.at[pl.ds(off, CHUNK)], dst_a)
                    pltpu.async_copy(fs[b].at[src_a], rows0, sem0)
                    pltpu.make_async_copy(fs[b].at[pl.ds(0, CHUNK)], rows1, sem1).wait()
                    pltpu.sync_copy(rows1, acc.at[dst_b], add=True)
                    return carry

                lax.fori_loop(0, EPC // 2, pair, 0)
                pltpu.make_async_copy(fs[b].at[pl.ds(0, CHUNK)], rows0, sem0).wait()
                plsc.subcore_barrier()
                r0 = sid * ZROWS
                for nz in ZCHUNKS:
                    pltpu.sync_copy(acc.at[pl.ds(r0, nz)],
                                    rows0.at[pl.ds(0, nz)])
                    pltpu.sync_copy(rows0.at[pl.ds(0, nz)],
                                    outs[b].at[pl.ds(r0, nz)])
                    r0 = r0 + nz

    return pl.kernel(
        body,
        out_type=[jax.ShapeDtypeStruct((ACC_ROWS, LANES), jnp.float32)
                  for _ in range(nb)],
        mesh=_sc_mesh(),
        scratch_types=[
            pltpu.VMEM((CHUNK,), jnp.int32),
            pltpu.VMEM((CHUNK,), jnp.int32),
            pltpu.VMEM((CHUNK,), jnp.int32),
            pltpu.VMEM((CHUNK,), jnp.int32),
            pltpu.VMEM((CHUNK, LANES), jnp.float32),
            pltpu.VMEM((CHUNK, LANES), jnp.float32),
            pltpu.VMEM((128, LANES), jnp.float32),
            pltpu.VMEM_SHARED((ACC_ROWS, LANES), jnp.float32),
            pltpu.SemaphoreType.DMA,
            pltpu.SemaphoreType.DMA,
        ],
        name=f"sage_segsum_{nb}",
    )


def _count_body(dst_hbm, z_hbm, ones_hbm, out0, out1, dst_a, dst_b, ones_v,
                zrows_v, rows_v, cacc, sem):
    cid = lax.axis_index("c")
    sid = lax.axis_index("s")
    outs = (out0, out1)
    pltpu.sync_copy(z_hbm, zrows_v)
    pltpu.sync_copy(ones_hbm, ones_v)
    r0 = sid * ZROWS
    for nz in ZCHUNKS:
        pltpu.sync_copy(zrows_v.at[pl.ds(0, nz)], cacc.at[pl.ds(r0, nz)])
        r0 = r0 + nz
    plsc.subcore_barrier()

    wpc = EPC // 2
    base = (cid * NS + sid) * wpc * CHUNK
    pltpu.sync_copy(dst_hbm.at[pl.ds(base, CHUNK)], dst_a)

    def pair(p, carry):
        pltpu.async_copy(ones_v, cacc.at[dst_a], sem, add=True)
        pltpu.sync_copy(dst_hbm.at[pl.ds(base + (2 * p + 1) * CHUNK, CHUNK)],
                        dst_b)
        pltpu.make_async_copy(z_hbm, ones_v, sem).wait()
        pltpu.async_copy(ones_v, cacc.at[dst_b], sem, add=True)
        off = base + ((2 * p + 2) % wpc) * CHUNK
        pltpu.sync_copy(dst_hbm.at[pl.ds(off, CHUNK)], dst_a)
        pltpu.make_async_copy(z_hbm, ones_v, sem).wait()
        return carry

    lax.fori_loop(0, wpc // 2, pair, 0)
    plsc.subcore_barrier()
    for c in range(2):
        @pl.when(cid == c)
        def _(c=c):
            r0 = sid * ZROWS
            for nz in ZCHUNKS:
                pltpu.sync_copy(cacc.at[pl.ds(r0, nz)],
                                rows_v.at[pl.ds(0, nz)])
                pltpu.sync_copy(rows_v.at[pl.ds(0, nz)],
                                outs[c].at[pl.ds(r0, nz)])
                r0 = r0 + nz


@functools.lru_cache(maxsize=None)
def _count_kernel():
    return pl.kernel(
        _count_body,
        out_type=[jax.ShapeDtypeStruct((ACC_ROWS, LANES), jnp.float32)
                  for _ in range(2)],
        mesh=_sc_mesh(),
        scratch_types=[
            pltpu.VMEM((CHUNK,), jnp.int32),
            pltpu.VMEM((CHUNK,), jnp.int32),
            pltpu.VMEM((CHUNK, LANES), jnp.float32),
            pltpu.VMEM((128, LANES), jnp.float32),
            pltpu.VMEM((128, LANES), jnp.float32),
            pltpu.VMEM_SHARED((ACC_ROWS, LANES), jnp.float32),
            pltpu.SemaphoreType.DMA,
        ],
        name="sage_counts",
    )


@functools.lru_cache(maxsize=None)
def _tc_layer(d_in, d_out, relu):
    def body(agg_ref, ca_ref, cb_ref, x_ref, wl_ref, wr_ref, b_ref, out_ref):
        cnt = ca_ref[...][:, 0:1] + cb_ref[...][:, 0:1]
        inv = 1.0 / jnp.maximum(cnt, 1.0)
        mean = agg_ref[...] * inv
        acc = jnp.dot(mean, wl_ref[...], preferred_element_type=jnp.float32)
        acc = acc + jnp.dot(x_ref[...], wr_ref[...],
                            preferred_element_type=jnp.float32)
        acc = acc + b_ref[...]
        out_ref[...] = jnp.maximum(acc, 0.0) if relu else acc

    return pl.pallas_call(
        body,
        grid=(N // MT,),
        in_specs=[
            pl.BlockSpec((MT, d_in), lambda i: (i, 0)),
            pl.BlockSpec((MT, 16), lambda i: (i, 0)),
            pl.BlockSpec((MT, 16), lambda i: (i, 0)),
            pl.BlockSpec((MT, d_in), lambda i: (i, 0)),
            pl.BlockSpec((d_in, d_out), lambda i: (0, 0)),
            pl.BlockSpec((d_in, d_out), lambda i: (0, 0)),
            pl.BlockSpec((1, d_out), lambda i: (0, 0)),
        ],
        out_specs=pl.BlockSpec((MT, d_out), lambda i: (i, 0)),
        out_shape=jax.ShapeDtypeStruct((N, d_out), jnp.float32),
    )


def kernel(x, edge_index, W1l, W1r, b1, W2l, W2r, b2, W3l, W3r, b3):
    src = edge_index[0].astype(jnp.int32)
    dst = edge_index[1].astype(jnp.int32)
    pad = E_PAD - E
    src = jnp.concatenate([src, jnp.zeros((pad,), jnp.int32)])
    dst = jnp.concatenate(
        [dst, N + (jnp.arange(pad, dtype=jnp.int32) % (ACC_ROWS - N))])
    z128 = jnp.zeros((128, LANES), jnp.float32)
    ones128 = jnp.ones((CHUNK, LANES), jnp.float32)
    cnt_a, cnt_b = _count_kernel()(dst, z128, ones128)
    cnt_a = cnt_a[:N, :16]
    cnt_b = cnt_b[:N, :16]

    h = x
    for wl, wr, b, relu in ((W1l, W1r, b1, True), (W2l, W2r, b2, True),
                            (W3l, W3r, b3, False)):
        d = h.shape[1]
        nb = d // LANES
        blocks = [lax.slice_in_dim(h, i * LANES, (i + 1) * LANES, axis=1)
                  for i in range(nb)]
        aggs = _segsum(nb)(*blocks, src, dst, z128)
        agg = jnp.concatenate([a[:N] for a in aggs], axis=1)
        h = _tc_layer(d, wl.shape[1], relu)(agg, cnt_a, cnt_b, h, wl, wr,
                                            b.reshape(1, -1))
    return h


# speedup vs baseline: 2.9524x; 1.0427x over previous
import functools

import jax
import jax.numpy as jnp
from jax import lax
from jax.experimental import pallas as pl
from jax.experimental.pallas import tpu as pltpu
from jax.experimental.pallas import tpu_sc as plsc

N = 10000
E = 160000
LANES = 128
CHUNK = 128
NS = 16
EPC = 80
EPT = EPC * CHUNK
E_PAD = EPT * NS
ACC_ROWS = 10112
ZROWS = ACC_ROWS // NS
ZCHUNKS = (128, 128, 128, 128, 120)
MT = 2000


def _sc_mesh():
    return plsc.VectorSubcoreMesh(core_axis_name="c", subcore_axis_name="s")


@functools.lru_cache(maxsize=None)
def _segsum(nb):
    bpc = nb // 2

    ngrp = 26

    def body(*refs):
        fs = refs[:nb]
        src_hbm, dst_hbm = refs[nb:nb + 2]
        outs = refs[nb + 2:nb + 2 + nb]
        (r0_, r1_, r2_, s0_, s1_, s2_, d0_, d1_, d2_, acc,
         sg0, sg1, sg2, ss0, ss1, ss2, si0, si1, si2) = refs[nb + 2 + nb:]
        rows = (r0_, r1_, r2_)
        srcv = (s0_, s1_, s2_)
        dstv = (d0_, d1_, d2_)
        sg = (sg0, sg1, sg2)
        ss = (ss0, ss1, ss2)
        si = (si0, si1, si2)
        cid = lax.axis_index("c")
        sid = lax.axis_index("s")
        zvec = jnp.zeros((16,), jnp.float32)

        def zero_rows0(i, carry):
            for j in range(8):
                r0_[i, pl.ds(j * 16, 16)] = zvec
            return carry

        lax.fori_loop(0, CHUNK, zero_rows0, 0)
        base = sid * EPT

        def wait_gather(k):
            pltpu.make_async_copy(fs[0].at[pl.ds(0, CHUNK)], rows[k],
                                  sg[k]).wait()

        def wait_scatter(k):
            pltpu.make_async_copy(fs[0].at[pl.ds(0, CHUNK)], rows[k],
                                  ss[k]).wait()

        def load_idx(k, g, sem):
            off = base + g * CHUNK
            pltpu.async_copy(src_hbm.at[pl.ds(off, CHUNK)], srcv[k], sem)
            pltpu.async_copy(dst_hbm.at[pl.ds(off, CHUNK)], dstv[k], sem)

        def wait_idx(k):
            pltpu.make_async_copy(src_hbm.at[pl.ds(0, CHUNK)], srcv[k],
                                  si[k]).wait()
            pltpu.make_async_copy(dst_hbm.at[pl.ds(0, CHUNK)], dstv[k],
                                  si[k]).wait()

        for b in range(nb):
            @pl.when(cid == b // bpc)
            def _(b=b):
                r0 = sid * ZROWS
                for nz in ZCHUNKS:
                    pltpu.sync_copy(r0_.at[pl.ds(0, nz)],
                                    acc.at[pl.ds(r0, nz)])
                    r0 = r0 + nz
                plsc.subcore_barrier()

                for k in range(3):
                    load_idx(k, k, si[k])
                for k in range(3):
                    wait_idx(k)
                    pltpu.async_copy(fs[b].at[srcv[k]], rows[k], sg[k])

                def group(q, carry):
                    for k in range(3):
                        wait_gather(k)
                        pltpu.async_copy(rows[k], acc.at[dstv[k]], ss[k],
                                         add=True)
                    for k in range(3):
                        wait_scatter(k)
                        load_idx(k, (3 * q + 3 + k) % EPC, si[k])
                    for k in range(3):
                        wait_idx(k)
                        pltpu.async_copy(fs[b].at[srcv[k]], rows[k], sg[k])
                    return carry

                lax.fori_loop(0, ngrp, group, 0)
                for k in range(2):
                    wait_gather(k)
                    pltpu.async_copy(rows[k], acc.at[dstv[k]], ss[k],
                                     add=True)
                wait_gather(2)
                for k in range(2):
                    wait_scatter(k)
                plsc.subcore_barrier()
                r0 = sid * ZROWS
                for nz in ZCHUNKS:
                    pltpu.sync_copy(acc.at[pl.ds(r0, nz)],
                                    r0_.at[pl.ds(0, nz)])
                    pltpu.sync_copy(r0_.at[pl.ds(0, nz)],
                                    outs[b].at[pl.ds(r0, nz)])
                    r0 = r0 + nz
                lax.fori_loop(0, CHUNK, zero_rows0, 0)

    return pl.kernel(
        body,
        out_type=[jax.ShapeDtypeStruct((ACC_ROWS, LANES), jnp.float32)
                  for _ in range(nb)],
        mesh=_sc_mesh(),
        scratch_types=[
            pltpu.VMEM((CHUNK, LANES), jnp.float32),
            pltpu.VMEM((CHUNK, LANES), jnp.float32),
            pltpu.VMEM((CHUNK, LANES), jnp.float32),
            pltpu.VMEM((CHUNK,), jnp.int32),
            pltpu.VMEM((CHUNK,), jnp.int32),
            pltpu.VMEM((CHUNK,), jnp.int32),
            pltpu.VMEM((CHUNK,), jnp.int32),
            pltpu.VMEM((CHUNK,), jnp.int32),
            pltpu.VMEM((CHUNK,), jnp.int32),
            pltpu.VMEM_SHARED((ACC_ROWS, LANES), jnp.float32),
            pltpu.SemaphoreType.DMA,
            pltpu.SemaphoreType.DMA,
            pltpu.SemaphoreType.DMA,
            pltpu.SemaphoreType.DMA,
            pltpu.SemaphoreType.DMA,
            pltpu.SemaphoreType.DMA,
            pltpu.SemaphoreType.DMA,
            pltpu.SemaphoreType.DMA,
            pltpu.SemaphoreType.DMA,
        ],
        name=f"sage_segsum_{nb}",
    )


def _count_body(dst_hbm, z_hbm, ones_hbm, out0, out1, dst_a, dst_b, ones_v,
                zrows_v, rows_v, cacc, sem):
    cid = lax.axis_index("c")
    sid = lax.axis_index("s")
    outs = (out0, out1)
    pltpu.sync_copy(z_hbm, zrows_v)
    pltpu.sync_copy(ones_hbm, ones_v)
    r0 = sid * ZROWS
    for nz in ZCHUNKS:
        pltpu.sync_copy(zrows_v.at[pl.ds(0, nz)], cacc.at[pl.ds(r0, nz)])
        r0 = r0 + nz
    plsc.subcore_barrier()

    wpc = EPC // 2
    base = (cid * NS + sid) * wpc * CHUNK
    pltpu.sync_copy(dst_hbm.at[pl.ds(base, CHUNK)], dst_a)

    def pair(p, carry):
        pltpu.async_copy(ones_v, cacc.at[dst_a], sem, add=True)
        pltpu.sync_copy(dst_hbm.at[pl.ds(base + (2 * p + 1) * CHUNK, CHUNK)],
                        dst_b)
        pltpu.make_async_copy(z_hbm, ones_v, sem).wait()
        pltpu.async_copy(ones_v, cacc.at[dst_b], sem, add=True)
        off = base + ((2 * p + 2) % wpc) * CHUNK
        pltpu.sync_copy(dst_hbm.at[pl.ds(off, CHUNK)], dst_a)
        pltpu.make_async_copy(z_hbm, ones_v, sem).wait()
        return carry

    lax.fori_loop(0, wpc // 2, pair, 0)
    plsc.subcore_barrier()
    for c in range(2):
        @pl.when(cid == c)
        def _(c=c):
            r0 = sid * ZROWS
            for nz in ZCHUNKS:
                pltpu.sync_copy(cacc.at[pl.ds(r0, nz)],
                                rows_v.at[pl.ds(0, nz)])
                pltpu.sync_copy(rows_v.at[pl.ds(0, nz)],
                                outs[c].at[pl.ds(r0, nz)])
                r0 = r0 + nz


@functools.lru_cache(maxsize=None)
def _count_kernel():
    return pl.kernel(
        _count_body,
        out_type=[jax.ShapeDtypeStruct((ACC_ROWS, LANES), jnp.float32)
                  for _ in range(2)],
        mesh=_sc_mesh(),
        scratch_types=[
            pltpu.VMEM((CHUNK,), jnp.int32),
            pltpu.VMEM((CHUNK,), jnp.int32),
            pltpu.VMEM((CHUNK, LANES), jnp.float32),
            pltpu.VMEM((128, LANES), jnp.float32),
            pltpu.VMEM((128, LANES), jnp.float32),
            pltpu.VMEM_SHARED((ACC_ROWS, LANES), jnp.float32),
            pltpu.SemaphoreType.DMA,
        ],
        name="sage_counts",
    )


@functools.lru_cache(maxsize=None)
def _tc_layer(d_in, d_out, relu):
    def body(agg_ref, ca_ref, cb_ref, x_ref, wl_ref, wr_ref, b_ref, out_ref):
        cnt = ca_ref[...][:, 0:1] + cb_ref[...][:, 0:1]
        inv = 1.0 / jnp.maximum(cnt, 1.0)
        mean = agg_ref[...] * inv
        acc = jnp.dot(mean, wl_ref[...], preferred_element_type=jnp.float32)
        acc = acc + jnp.dot(x_ref[...], wr_ref[...],
                            preferred_element_type=jnp.float32)
        acc = acc + b_ref[...]
        out_ref[...] = jnp.maximum(acc, 0.0) if relu else acc

    return pl.pallas_call(
        body,
        grid=(N // MT,),
        in_specs=[
            pl.BlockSpec((MT, d_in), lambda i: (i, 0)),
            pl.BlockSpec((MT, 16), lambda i: (i, 0)),
            pl.BlockSpec((MT, 16), lambda i: (i, 0)),
            pl.BlockSpec((MT, d_in), lambda i: (i, 0)),
            pl.BlockSpec((d_in, d_out), lambda i: (0, 0)),
            pl.BlockSpec((d_in, d_out), lambda i: (0, 0)),
            pl.BlockSpec((1, d_out), lambda i: (0, 0)),
        ],
        out_specs=pl.BlockSpec((MT, d_out), lambda i: (i, 0)),
        out_shape=jax.ShapeDtypeStruct((N, d_out), jnp.float32),
    )


def kernel(x, edge_index, W1l, W1r, b1, W2l, W2r, b2, W3l, W3r, b3):
    src = edge_index[0].astype(jnp.int32)
    dst = edge_index[1].astype(jnp.int32)
    pad = E_PAD - E
    src = jnp.concatenate([src, jnp.zeros((pad,), jnp.int32)])
    dst = jnp.concatenate(
        [dst, N + (jnp.arange(pad, dtype=jnp.int32) % (ACC_ROWS - N))])
    z128 = jnp.zeros((128, LANES), jnp.float32)
    ones128 = jnp.ones((CHUNK, LANES), jnp.float32)
    cnt_a, cnt_b = _count_kernel()(dst, z128, ones128)
    cnt_a = cnt_a[:N, :16]
    cnt_b = cnt_b[:N, :16]

    h = x
    for wl, wr, b, relu in ((W1l, W1r, b1, True), (W2l, W2r, b2, True),
                            (W3l, W3r, b3, False)):
        d = h.shape[1]
        nb = d // LANES
        blocks = [lax.slice_in_dim(h, i * LANES, (i + 1) * LANES, axis=1)
                  for i in range(nb)]
        aggs = _segsum(nb)(*blocks, src, dst)
        agg = jnp.concatenate([a[:N] for a in aggs], axis=1)
        h = _tc_layer(d, wl.shape[1], relu)(agg, cnt_a, cnt_b, h, wl, wr,
                                            b.reshape(1, -1))
    return h


# speedup vs baseline: 3.4871x; 1.1811x over previous
import functools

import jax
import jax.numpy as jnp
from jax import lax
from jax.experimental import pallas as pl
from jax.experimental.pallas import tpu as pltpu
from jax.experimental.pallas import tpu_sc as plsc

N = 10000
E = 160000
D_HID = 512
D_OUT = 256
LANES = 128
CHUNK = 128
NS = 16
EPC = 80
EPT = EPC * CHUNK
E_PAD = EPT * NS
ACC_ROWS = 10112
ZROWS = ACC_ROWS // NS
ZCHUNKS = (128, 128, 128, 128, 120)
MT = 2000


def _sc_mesh():
    return plsc.VectorSubcoreMesh(core_axis_name="c", subcore_axis_name="s")


@functools.lru_cache(maxsize=None)
def _segsum(nb):
    bpc = nb // 2

    ngrp = 26

    def body(*refs):
        fs = refs[:nb]
        src_hbm, dst_hbm = refs[nb:nb + 2]
        outs = refs[nb + 2:nb + 2 + nb]
        (r0_, r1_, r2_, s0_, s1_, s2_, d0_, d1_, d2_, acc,
         sg0, sg1, sg2, ss0, ss1, ss2, si0, si1, si2) = refs[nb + 2 + nb:]
        rows = (r0_, r1_, r2_)
        srcv = (s0_, s1_, s2_)
        dstv = (d0_, d1_, d2_)
        sg = (sg0, sg1, sg2)
        ss = (ss0, ss1, ss2)
        si = (si0, si1, si2)
        cid = lax.axis_index("c")
        sid = lax.axis_index("s")
        zvec = jnp.zeros((16,), jnp.float32)

        def zero_rows0(i, carry):
            for j in range(8):
                r0_[i, pl.ds(j * 16, 16)] = zvec
            return carry

        lax.fori_loop(0, CHUNK, zero_rows0, 0)
        base = sid * EPT

        def wait_gather(k):
            pltpu.make_async_copy(fs[0].at[pl.ds(0, CHUNK)], rows[k],
                                  sg[k]).wait()

        def wait_scatter(k):
            pltpu.make_async_copy(fs[0].at[pl.ds(0, CHUNK)], rows[k],
                                  ss[k]).wait()

        def load_idx(k, g, sem):
            off = base + g * CHUNK
            pltpu.async_copy(src_hbm.at[pl.ds(off, CHUNK)], srcv[k], sem)
            pltpu.async_copy(dst_hbm.at[pl.ds(off, CHUNK)], dstv[k], sem)

        def wait_idx(k):
            pltpu.make_async_copy(src_hbm.at[pl.ds(0, CHUNK)], srcv[k],
                                  si[k]).wait()
            pltpu.make_async_copy(dst_hbm.at[pl.ds(0, CHUNK)], dstv[k],
                                  si[k]).wait()

        for b in range(nb):
            @pl.when(cid == b // bpc)
            def _(b=b):
                r0 = sid * ZROWS
                for nz in ZCHUNKS:
                    pltpu.sync_copy(r0_.at[pl.ds(0, nz)],
                                    acc.at[pl.ds(r0, nz)])
                    r0 = r0 + nz
                plsc.subcore_barrier()

                for k in range(3):
                    load_idx(k, k, si[k])
                for k in range(3):
                    wait_idx(k)
                    pltpu.async_copy(fs[b].at[srcv[k]], rows[k], sg[k])

                def group(q, carry):
                    for k in range(3):
                        wait_gather(k)
                        pltpu.async_copy(rows[k], acc.at[dstv[k]], ss[k],
                                         add=True)
                    for k in range(3):
                        wait_scatter(k)
                        load_idx(k, (3 * q + 3 + k) % EPC, si[k])
                    for k in range(3):
                        wait_idx(k)
                        pltpu.async_copy(fs[b].at[srcv[k]], rows[k], sg[k])
                    return carry

                lax.fori_loop(0, ngrp, group, 0)
                for k in range(2):
                    wait_gather(k)
                    pltpu.async_copy(rows[k], acc.at[dstv[k]], ss[k],
                                     add=True)
                wait_gather(2)
                for k in range(2):
                    wait_scatter(k)
                plsc.subcore_barrier()
                r0 = sid * ZROWS
                for nz in ZCHUNKS:
                    pltpu.sync_copy(acc.at[pl.ds(r0, nz)],
                                    r0_.at[pl.ds(0, nz)])
                    pltpu.sync_copy(r0_.at[pl.ds(0, nz)],
                                    outs[b].at[pl.ds(r0, nz)])
                    r0 = r0 + nz
                lax.fori_loop(0, CHUNK, zero_rows0, 0)

    return pl.kernel(
        body,
        out_type=[jax.ShapeDtypeStruct((ACC_ROWS, LANES), jnp.float32)
                  for _ in range(nb)],
        mesh=_sc_mesh(),
        scratch_types=[
            pltpu.VMEM((CHUNK, LANES), jnp.float32),
            pltpu.VMEM((CHUNK, LANES), jnp.float32),
            pltpu.VMEM((CHUNK, LANES), jnp.float32),
            pltpu.VMEM((CHUNK,), jnp.int32),
            pltpu.VMEM((CHUNK,), jnp.int32),
            pltpu.VMEM((CHUNK,), jnp.int32),
            pltpu.VMEM((CHUNK,), jnp.int32),
            pltpu.VMEM((CHUNK,), jnp.int32),
            pltpu.VMEM((CHUNK,), jnp.int32),
            pltpu.VMEM_SHARED((ACC_ROWS, LANES), jnp.float32),
            pltpu.SemaphoreType.DMA,
            pltpu.SemaphoreType.DMA,
            pltpu.SemaphoreType.DMA,
            pltpu.SemaphoreType.DMA,
            pltpu.SemaphoreType.DMA,
            pltpu.SemaphoreType.DMA,
            pltpu.SemaphoreType.DMA,
            pltpu.SemaphoreType.DMA,
            pltpu.SemaphoreType.DMA,
        ],
        name=f"sage_segsum_{nb}",
    )


def _count_body(dst_hbm, z_hbm, ones_hbm, out0, out1, dst_a, dst_b, ones_v,
                zrows_v, rows_v, cacc, sem):
    cid = lax.axis_index("c")
    sid = lax.axis_index("s")
    outs = (out0, out1)
    pltpu.sync_copy(z_hbm, zrows_v)
    pltpu.sync_copy(ones_hbm, ones_v)
    r0 = sid * ZROWS
    for nz in ZCHUNKS:
        pltpu.sync_copy(zrows_v.at[pl.ds(0, nz)], cacc.at[pl.ds(r0, nz)])
        r0 = r0 + nz
    plsc.subcore_barrier()

    wpc = EPC // 2
    base = (cid * NS + sid) * wpc * CHUNK
    pltpu.sync_copy(dst_hbm.at[pl.ds(base, CHUNK)], dst_a)

    def pair(p, carry):
        pltpu.async_copy(ones_v, cacc.at[dst_a], sem, add=True)
        pltpu.sync_copy(dst_hbm.at[pl.ds(base + (2 * p + 1) * CHUNK, CHUNK)],
                        dst_b)
        pltpu.make_async_copy(z_hbm, ones_v, sem).wait()
        pltpu.async_copy(ones_v, cacc.at[dst_b], sem, add=True)
        off = base + ((2 * p + 2) % wpc) * CHUNK
        pltpu.sync_copy(dst_hbm.at[pl.ds(off, CHUNK)], dst_a)
        pltpu.make_async_copy(z_hbm, ones_v, sem).wait()
        return carry

    lax.fori_loop(0, wpc // 2, pair, 0)
    plsc.subcore_barrier()
    for c in range(2):
        @pl.when(cid == c)
        def _(c=c):
            r0 = sid * ZROWS
            for nz in ZCHUNKS:
                pltpu.sync_copy(cacc.at[pl.ds(r0, nz)],
                                rows_v.at[pl.ds(0, nz)])
                pltpu.sync_copy(rows_v.at[pl.ds(0, nz)],
                                outs[c].at[pl.ds(r0, nz)])
                r0 = r0 + nz


@functools.lru_cache(maxsize=None)
def _count_kernel():
    return pl.kernel(
        _count_body,
        out_type=[jax.ShapeDtypeStruct((ACC_ROWS, LANES), jnp.float32)
                  for _ in range(2)],
        mesh=_sc_mesh(),
        scratch_types=[
            pltpu.VMEM((CHUNK,), jnp.int32),
            pltpu.VMEM((CHUNK,), jnp.int32),
            pltpu.VMEM((CHUNK, LANES), jnp.float32),
            pltpu.VMEM((128, LANES), jnp.float32),
            pltpu.VMEM((128, LANES), jnp.float32),
            pltpu.VMEM_SHARED((ACC_ROWS, LANES), jnp.float32),
            pltpu.SemaphoreType.DMA,
        ],
        name="sage_counts",
    )


@functools.lru_cache(maxsize=None)
def _tc_layer(d_in, d_out, relu):
    def body(agg_ref, ca_ref, cb_ref, x_ref, wl_ref, wr_ref, b_ref, out_ref):
        cnt = ca_ref[...][:, 0:1] + cb_ref[...][:, 0:1]
        inv = 1.0 / jnp.maximum(cnt, 1.0)
        mean = agg_ref[...] * inv
        acc = jnp.dot(mean, wl_ref[...], preferred_element_type=jnp.float32)
        acc = acc + jnp.dot(x_ref[...], wr_ref[...],
                            preferred_element_type=jnp.float32)
        acc = acc + b_ref[...]
        out_ref[...] = jnp.maximum(acc, 0.0) if relu else acc

    return pl.pallas_call(
        body,
        grid=(N // MT,),
        in_specs=[
            pl.BlockSpec((MT, d_in), lambda i: (i, 0)),
            pl.BlockSpec((MT, 16), lambda i: (i, 0)),
            pl.BlockSpec((MT, 16), lambda i: (i, 0)),
            pl.BlockSpec((MT, d_in), lambda i: (i, 0)),
            pl.BlockSpec((d_in, d_out), lambda i: (0, 0)),
            pl.BlockSpec((d_in, d_out), lambda i: (0, 0)),
            pl.BlockSpec((1, d_out), lambda i: (0, 0)),
        ],
        out_specs=pl.BlockSpec((MT, d_out), lambda i: (i, 0)),
        out_shape=jax.ShapeDtypeStruct((N, d_out), jnp.float32),
    )


@functools.lru_cache(maxsize=None)
def _tc_matmul(d_in, d_out):
    def body(x_ref, w_ref, b_ref, out_ref):
        out_ref[...] = jnp.dot(x_ref[...], w_ref[...],
                               preferred_element_type=jnp.float32) + b_ref[...]

    return pl.pallas_call(
        body,
        grid=(N // MT,),
        in_specs=[
            pl.BlockSpec((MT, d_in), lambda i: (i, 0)),
            pl.BlockSpec((d_in, d_out), lambda i: (0, 0)),
            pl.BlockSpec((1, d_out), lambda i: (0, 0)),
        ],
        out_specs=pl.BlockSpec((MT, d_out), lambda i: (i, 0)),
        out_shape=jax.ShapeDtypeStruct((N, d_out), jnp.float32),
    )


@functools.lru_cache(maxsize=None)
def _tc_post(d):
    def body(agg_ref, ca_ref, cb_ref, z_ref, out_ref):
        cnt = ca_ref[...][:, 0:1] + cb_ref[...][:, 0:1]
        inv = 1.0 / jnp.maximum(cnt, 1.0)
        out_ref[...] = agg_ref[...] * inv + z_ref[...]

    return pl.pallas_call(
        body,
        grid=(N // MT,),
        in_specs=[
            pl.BlockSpec((MT, d), lambda i: (i, 0)),
            pl.BlockSpec((MT, 16), lambda i: (i, 0)),
            pl.BlockSpec((MT, 16), lambda i: (i, 0)),
            pl.BlockSpec((MT, d), lambda i: (i, 0)),
        ],
        out_specs=pl.BlockSpec((MT, d), lambda i: (i, 0)),
        out_shape=jax.ShapeDtypeStruct((N, d), jnp.float32),
    )


def kernel(x, edge_index, W1l, W1r, b1, W2l, W2r, b2, W3l, W3r, b3):
    src = edge_index[0].astype(jnp.int32)
    dst = edge_index[1].astype(jnp.int32)
    pad = E_PAD - E
    src = jnp.concatenate([src, jnp.zeros((pad,), jnp.int32)])
    dst = jnp.concatenate(
        [dst, N + (jnp.arange(pad, dtype=jnp.int32) % (ACC_ROWS - N))])
    z128 = jnp.zeros((128, LANES), jnp.float32)
    ones128 = jnp.ones((CHUNK, LANES), jnp.float32)
    cnt_a, cnt_b = _count_kernel()(dst, z128, ones128)
    cnt_a = cnt_a[:N, :16]
    cnt_b = cnt_b[:N, :16]

    def seg(h, nb):
        blocks = [lax.slice_in_dim(h, i * LANES, (i + 1) * LANES, axis=1)
                  for i in range(nb)]
        aggs = _segsum(nb)(*blocks, src, dst)
        return jnp.concatenate([a[:N] for a in aggs], axis=1)

    h = x
    for wl, wr, b in ((W1l, W1r, b1), (W2l, W2r, b2)):
        d = h.shape[1]
        agg = seg(h, d // LANES)
        h = _tc_layer(d, wl.shape[1], True)(agg, cnt_a, cnt_b, h, wl, wr,
                                            b.reshape(1, -1))
    wcat = jnp.concatenate([W3l, W3r], axis=1)
    bcat = jnp.concatenate([jnp.zeros_like(b3), b3]).reshape(1, -1)
    yz = _tc_matmul(D_HID, 2 * D_OUT)(h, wcat, bcat)
    y = yz[:, :D_OUT]
    z = yz[:, D_OUT:]
    agg3 = seg(y, D_OUT // LANES)
    return _tc_post(D_OUT)(agg3, cnt_a, cnt_b, z)
